# baseline (speedup 1.0000x reference)
import functools

import jax
import jax.numpy as jnp
from jax.experimental import pallas as pl
from jax.experimental.pallas import tpu as pltpu

_EPS = 1e-8


def _body(w_ref, m_ref, keys_ref, vals_ref, out_ref, best_s_ref, best_i_ref,
          sem, *, rows):
    i = pl.program_id(0)

    @pl.when(i == 0)
    def _init():
        best_s_ref[0] = -jnp.inf
        best_i_ref[0] = 0

    b = keys_ref[...]
    dn = (((1,), (0,)), ((), ()))
    dots = jax.lax.dot_general(b, w_ref[...], dn,
                               precision=jax.lax.Precision.HIGHEST,
                               preferred_element_type=jnp.float32)
    sumsq = jax.lax.dot_general(b * b, m_ref[...], dn,
                                precision=jax.lax.Precision.HIGHEST,
                                preferred_element_type=jnp.float32)
    scores = dots / jnp.maximum(jnp.sqrt(sumsq), _EPS)
    gidx = (jax.lax.broadcasted_iota(jnp.int32, scores.shape, 0) * 2
            + jax.lax.broadcasted_iota(jnp.int32, scores.shape, 1)
            + i * (2 * rows))
    local_max = jnp.max(scores)
    local_arg = jnp.min(jnp.where(scores == local_max, gidx,
                                  jnp.int32(2147483647)))

    @pl.when(local_max > best_s_ref[0])
    def _update():
        best_s_ref[0] = local_max
        best_i_ref[0] = local_arg

    @pl.when(i == pl.num_programs(0) - 1)
    def _gather():
        idx = best_i_ref[0]
        cp = pltpu.make_async_copy(vals_ref.at[pl.ds(idx, 1), :], out_ref, sem)
        cp.start()
        cp.wait()


def _pick_rows(n2):
    for r in (10000, 8000, 6250, 5000, 4000, 2500, 2000, 1250, 1000, 625, 500,
              250, 200, 125, 100, 50, 25, 20, 10, 8, 5, 4, 2, 1):
        if r <= n2 and n2 % r == 0:
            return r
    return 1


def kernel(key, keys, vals):
    n, d = keys.shape
    n2 = n // 2
    keys2 = keys.reshape(n2, 2 * d)
    rows = _pick_rows(n2)

    zcol = jnp.zeros((d, 1), jnp.float32)
    kcol = key.reshape(d, 1).astype(jnp.float32)
    ocol = jnp.ones((d, 1), jnp.float32)
    w = jnp.concatenate(
        [jnp.concatenate([kcol, zcol], axis=1),
         jnp.concatenate([zcol, kcol], axis=1)], axis=0)
    m = jnp.concatenate(
        [jnp.concatenate([ocol, zcol], axis=1),
         jnp.concatenate([zcol, ocol], axis=1)], axis=0)

    out = pl.pallas_call(
        functools.partial(_body, rows=rows),
        grid=(n2 // rows,),
        in_specs=[
            pl.BlockSpec((2 * d, 2), lambda i: (0, 0)),
            pl.BlockSpec((2 * d, 2), lambda i: (0, 0)),
            pl.BlockSpec((rows, 2 * d), lambda i: (i, 0)),
            pl.BlockSpec(memory_space=pltpu.MemorySpace.HBM),
        ],
        out_specs=pl.BlockSpec(memory_space=pltpu.MemorySpace.HBM),
        out_shape=jax.ShapeDtypeStruct((1, vals.shape[1]), jnp.float32),
        scratch_shapes=[
            pltpu.SMEM((1,), jnp.float32),
            pltpu.SMEM((1,), jnp.int32),
            pltpu.SemaphoreType.DMA,
        ],
    )(w, m, keys2, vals)
    return out.reshape(vals.shape[1])


# speedup vs baseline: 1.1381x; 1.1381x over previous
import functools

import jax
import jax.numpy as jnp
from jax.experimental import pallas as pl
from jax.experimental.pallas import tpu as pltpu

_EPS = 1e-8


def _body(w_ref, m_ref, keys_ref, vals_ref, out_ref, best_s_ref, best_i_ref,
          sem, *, rows):
    i = pl.program_id(0)

    @pl.when(i == 0)
    def _init():
        best_s_ref[0] = -jnp.inf
        best_i_ref[0] = 0

    b = keys_ref[...]
    dn = (((1,), (1,)), ((), ()))
    dots = jax.lax.dot_general(w_ref[...], b, dn,
                               precision=jax.lax.Precision.HIGHEST,
                               preferred_element_type=jnp.float32)
    sumsq = jax.lax.dot_general(m_ref[...], b * b, dn,
                                precision=jax.lax.Precision.HIGHEST,
                                preferred_element_type=jnp.float32)
    scores = dots / jnp.maximum(jnp.sqrt(sumsq), _EPS)
    gidx = (jax.lax.broadcasted_iota(jnp.int32, scores.shape, 1) * 2
            + jax.lax.broadcasted_iota(jnp.int32, scores.shape, 0)
            + i * (2 * rows))
    local_max = jnp.max(scores)
    local_arg = jnp.min(jnp.where(scores == local_max, gidx,
                                  jnp.int32(2147483647)))

    @pl.when(local_max > best_s_ref[0])
    def _update():
        best_s_ref[0] = local_max
        best_i_ref[0] = local_arg

    @pl.when(i == pl.num_programs(0) - 1)
    def _gather():
        idx = best_i_ref[0]
        cp = pltpu.make_async_copy(vals_ref.at[pl.ds(idx, 1), :], out_ref, sem)
        cp.start()
        cp.wait()


def _pick_rows(n2):
    for r in (10000, 8000, 6250, 5000, 4000, 2500, 2000, 1250, 1000, 625, 500,
              250, 200, 125, 100, 50, 25, 20, 10, 8, 5, 4, 2, 1):
        if r <= n2 and n2 % r == 0:
            return r
    return 1


def kernel(key, keys, vals):
    n, d = keys.shape
    n2 = n // 2
    keys2 = keys.reshape(n2, 2 * d)
    rows = _pick_rows(n2)

    zrow = jnp.zeros((1, d), jnp.float32)
    krow = key.reshape(1, d).astype(jnp.float32)
    orow = jnp.ones((1, d), jnp.float32)
    w = jnp.concatenate(
        [jnp.concatenate([krow, zrow], axis=1),
         jnp.concatenate([zrow, krow], axis=1)], axis=0)
    m = jnp.concatenate(
        [jnp.concatenate([orow, zrow], axis=1),
         jnp.concatenate([zrow, orow], axis=1)], axis=0)

    out = pl.pallas_call(
        functools.partial(_body, rows=rows),
        grid=(n2 // rows,),
        in_specs=[
            pl.BlockSpec((2, 2 * d), lambda i: (0, 0)),
            pl.BlockSpec((2, 2 * d), lambda i: (0, 0)),
            pl.BlockSpec((rows, 2 * d), lambda i: (i, 0)),
            pl.BlockSpec(memory_space=pltpu.MemorySpace.HBM),
        ],
        out_specs=pl.BlockSpec(memory_space=pltpu.MemorySpace.HBM),
        out_shape=jax.ShapeDtypeStruct((1, vals.shape[1]), jnp.float32),
        scratch_shapes=[
            pltpu.SMEM((1,), jnp.float32),
            pltpu.SMEM((1,), jnp.int32),
            pltpu.SemaphoreType.DMA,
        ],
    )(w, m, keys2, vals)
    return out.reshape(vals.shape[1])


# speedup vs baseline: 1.5096x; 1.3265x over previous
import functools

import jax
import jax.numpy as jnp
from jax.experimental import pallas as pl
from jax.experimental.pallas import tpu as pltpu

_EPS = 1e-8


def _body(w_ref, m_ref, keys_ref, vals_ref, out_ref, best_s_ref, best_i_ref,
          sem, *, rows):
    i = pl.program_id(0)

    @pl.when(i == 0)
    def _init():
        best_s_ref[0] = -jnp.inf
        best_i_ref[0] = 0

    b = keys_ref[...]
    dn = (((1,), (1,)), ((), ()))
    dots = jax.lax.dot_general(w_ref[...], b, dn,
                               preferred_element_type=jnp.float32)
    sumsq = jax.lax.dot_general(m_ref[...], b * b, dn,
                                preferred_element_type=jnp.float32)
    scores = dots / jnp.maximum(jnp.sqrt(sumsq), _EPS)
    gidx = (jax.lax.broadcasted_iota(jnp.int32, scores.shape, 1) * 2
            + jax.lax.broadcasted_iota(jnp.int32, scores.shape, 0)
            + i * (2 * rows))
    local_max = jnp.max(scores)
    local_arg = jnp.min(jnp.where(scores == local_max, gidx,
                                  jnp.int32(2147483647)))

    @pl.when(local_max > best_s_ref[0])
    def _update():
        best_s_ref[0] = local_max
        best_i_ref[0] = local_arg

    @pl.when(i == pl.num_programs(0) - 1)
    def _gather():
        idx = best_i_ref[0]
        cp = pltpu.make_async_copy(vals_ref.at[pl.ds(idx, 1), :], out_ref, sem)
        cp.start()
        cp.wait()


def _pick_rows(n2):
    for r in (10000, 8000, 6250, 5000, 4000, 2500, 2000, 1250, 1000, 625, 500,
              250, 200, 125, 100, 50, 25, 20, 10, 8, 5, 4, 2, 1):
        if r <= n2 and n2 % r == 0:
            return r
    return 1


def kernel(key, keys, vals):
    n, d = keys.shape
    n2 = n // 2
    keys2 = keys.reshape(n2, 2 * d)
    rows = _pick_rows(n2)

    zrow = jnp.zeros((1, d), jnp.float32)
    krow = key.reshape(1, d).astype(jnp.float32)
    orow = jnp.ones((1, d), jnp.float32)
    w = jnp.concatenate(
        [jnp.concatenate([krow, zrow], axis=1),
         jnp.concatenate([zrow, krow], axis=1)], axis=0)
    m = jnp.concatenate(
        [jnp.concatenate([orow, zrow], axis=1),
         jnp.concatenate([zrow, orow], axis=1)], axis=0)

    out = pl.pallas_call(
        functools.partial(_body, rows=rows),
        grid=(n2 // rows,),
        in_specs=[
            pl.BlockSpec((2, 2 * d), lambda i: (0, 0)),
            pl.BlockSpec((2, 2 * d), lambda i: (0, 0)),
            pl.BlockSpec((rows, 2 * d), lambda i: (i, 0)),
            pl.BlockSpec(memory_space=pltpu.MemorySpace.HBM),
        ],
        out_specs=pl.BlockSpec(memory_space=pltpu.MemorySpace.HBM),
        out_shape=jax.ShapeDtypeStruct((1, vals.shape[1]), jnp.float32),
        scratch_shapes=[
            pltpu.SMEM((1,), jnp.float32),
            pltpu.SMEM((1,), jnp.int32),
            pltpu.SemaphoreType.DMA,
        ],
    )(w, m, keys2, vals)
    return out.reshape(vals.shape[1])


# speedup vs baseline: 1.7156x; 1.1364x over previous
import functools

import jax
import jax.numpy as jnp
from jax.experimental import pallas as pl
from jax.experimental.pallas import tpu as pltpu

_EPS = 1e-8


def _body(w_ref, m_ref, keys_ref, vals_ref, out_ref, best_s_ref, best_i_ref,
          sem, *, rows):
    i = pl.program_id(0)

    @pl.when(i == 0)
    def _init():
        best_s_ref[0] = -jnp.inf
        best_i_ref[0] = 0

    b = keys_ref[...]
    dn = (((1,), (1,)), ((), ()))
    dots = jax.lax.dot_general(w_ref[...], b, dn,
                               preferred_element_type=jnp.float32)
    sumsq = jax.lax.dot_general(m_ref[...], b * b, dn,
                                preferred_element_type=jnp.float32)
    scores = dots / jnp.maximum(jnp.sqrt(sumsq), _EPS)
    gidx = jax.lax.broadcasted_iota(jnp.int32, scores.shape, 1) + i * rows
    local_max = jnp.max(scores)
    local_arg = jnp.min(jnp.where(scores == local_max, gidx,
                                  jnp.int32(2147483647)))

    @pl.when(local_max > best_s_ref[0])
    def _update():
        best_s_ref[0] = local_max
        best_i_ref[0] = local_arg

    @pl.when(i == pl.num_programs(0) - 1)
    def _gather():
        idx = best_i_ref[0]
        cp = pltpu.make_async_copy(vals_ref.at[pl.ds(idx, 1), :], out_ref, sem)
        cp.start()
        cp.wait()


def _pick_rows(n):
    for r in (20000, 16000, 12500, 10000, 8000, 6250, 5000, 4000, 2500, 2000,
              1250, 1000, 625, 500, 250, 200, 125, 100, 50, 25, 20, 10, 8, 5,
              4, 2, 1):
        if r <= n and n % r == 0:
            return r
    return 1


def kernel(key, keys, vals):
    n, d = keys.shape
    rows = _pick_rows(n)

    w = key.reshape(1, d).astype(jnp.float32)
    m = jnp.ones((1, d), jnp.float32)

    out = pl.pallas_call(
        functools.partial(_body, rows=rows),
        grid=(n // rows,),
        in_specs=[
            pl.BlockSpec((1, d), lambda i: (0, 0)),
            pl.BlockSpec((1, d), lambda i: (0, 0)),
            pl.BlockSpec((rows, d), lambda i: (i, 0)),
            pl.BlockSpec(memory_space=pltpu.MemorySpace.HBM),
        ],
        out_specs=pl.BlockSpec(memory_space=pltpu.MemorySpace.HBM),
        out_shape=jax.ShapeDtypeStruct((1, vals.shape[1]), jnp.float32),
        scratch_shapes=[
            pltpu.SMEM((1,), jnp.float32),
            pltpu.SMEM((1,), jnp.int32),
            pltpu.SemaphoreType.DMA,
        ],
    )(w, m, keys, vals)
    return out.reshape(vals.shape[1])
